# Initial kernel scaffold; baseline (speedup 1.0000x reference)
import jax
import jax.numpy as jnp
from jax.experimental import pallas as pl


def kernel(x, emb, W, b):
    raise NotImplementedError("write your pallas kernel here")


# speedup vs baseline: 2.4913x; 2.4913x over previous
import functools

import jax
import jax.numpy as jnp
from jax import lax
from jax.experimental import pallas as pl
from jax.experimental.pallas import tpu as pltpu
from jax.experimental.pallas import tpu_sc as plsc

_VOCAB = 1000000
_D = 64
_OUT = 100
_B = 16384
_H = 50

_NC, _NS = 2, 16
_NW = _NC * _NS
_BAGS_W = _B // _NW
_PAIR = 2
_ROWS_CHUNK = _PAIR * _H
_NCHUNK = _BAGS_W // _PAIR
_NBUF = 2
_LG = _D // 16


def _sc_bag_sum(xr, emb):
    mesh = plsc.VectorSubcoreMesh(
        core_axis_name="c", subcore_axis_name="s",
        num_cores=_NC, num_subcores=_NS)

    @functools.partial(
        pl.kernel,
        out_type=jax.ShapeDtypeStruct((_B, _D), jnp.float32),
        mesh=mesh,
        compiler_params=pltpu.CompilerParams(use_tc_tiling_on_sc=False),
        scratch_types=[
            pltpu.VMEM((_NCHUNK, _ROWS_CHUNK), jnp.int32),
            [pltpu.VMEM((_ROWS_CHUNK, _D), jnp.float32) for _ in range(_NBUF)],
            pltpu.VMEM((_BAGS_W, _D), jnp.float32),
            [pltpu.SemaphoreType.DMA for _ in range(_NBUF)],
        ],
    )
    def k(x_hbm, emb_hbm, out_hbm, idx_v, bufs, bag_v, sems):
        wid = lax.axis_index("s") * _NC + lax.axis_index("c")
        pltpu.sync_copy(x_hbm.at[pl.ds(wid * _NCHUNK, _NCHUNK)], idx_v)
        for s in range(_NBUF):
            pltpu.async_copy(emb_hbm.at[idx_v.at[s]], bufs[s], sems[s])

        def accum(j, buf):
            for h in range(_PAIR):
                r0 = h * _H
                acc = [buf[r0, pl.ds(l * 16, 16)] for l in range(_LG)]
                for r in range(1, _H):
                    for l in range(_LG):
                        acc[l] = acc[l] + buf[r0 + r, pl.ds(l * 16, 16)]
                for l in range(_LG):
                    bag_v[j * _PAIR + h, pl.ds(l * 16, 16)] = acc[l]

        @pl.loop(0, _NCHUNK, step=_NBUF)
        def _(jbase):
            for s in range(_NBUF):
                j = jbase + s
                pltpu.make_async_copy(
                    emb_hbm.at[idx_v.at[j]], bufs[s], sems[s]).wait()
                accum(j, bufs[s])
                nxt = j + _NBUF

                @pl.when(nxt < _NCHUNK)
                def _():
                    pltpu.async_copy(
                        emb_hbm.at[idx_v.at[nxt]], bufs[s], sems[s])

        pltpu.sync_copy(bag_v, out_hbm.at[pl.ds(wid * _BAGS_W, _BAGS_W)])

    return k(xr, emb)


def _tc_head(bag, wp, bp):
    tb = 1024

    def body(bag_ref, w_ref, b_ref, out_ref):
        y = jnp.dot(bag_ref[...], w_ref[...],
                    preferred_element_type=jnp.float32) + b_ref[...]
        m = jnp.max(y, axis=1, keepdims=True)
        e = jnp.exp(y - m)
        out_ref[...] = e / jnp.sum(e, axis=1, keepdims=True)

    return pl.pallas_call(
        body,
        grid=(_B // tb,),
        in_specs=[
            pl.BlockSpec((tb, _D), lambda i: (i, 0)),
            pl.BlockSpec((_D, 128), lambda i: (0, 0)),
            pl.BlockSpec((1, 128), lambda i: (0, 0)),
        ],
        out_specs=pl.BlockSpec((tb, 128), lambda i: (i, 0)),
        out_shape=jax.ShapeDtypeStruct((_B, 128), jnp.float32),
    )(bag, wp, bp)


def kernel(x, emb, W, b):
    x = x.astype(jnp.int32)
    xr = x.reshape(_NW * _NCHUNK, _ROWS_CHUNK)
    bag = _sc_bag_sum(xr, emb)
    wp = jnp.zeros((_D, 128), jnp.float32).at[:, :_OUT].set(W.T * (1.0 / _H))
    bp = jnp.full((1, 128), -1e30, jnp.float32).at[0, :_OUT].set(b)
    out = _tc_head(bag, wp, bp)
    return out[:, :_OUT]


# speedup vs baseline: 2.4937x; 1.0009x over previous
import functools

import jax
import jax.numpy as jnp
from jax import lax
from jax.experimental import pallas as pl
from jax.experimental.pallas import tpu as pltpu
from jax.experimental.pallas import tpu_sc as plsc

_VOCAB = 1000000
_D = 64
_OUT = 100
_B = 16384
_H = 50

_NC, _NS = 2, 16
_NW = _NC * _NS
_BAGS_W = _B // _NW
_PAIR = 2
_ROWS_CHUNK = _PAIR * _H
_NCHUNK = _BAGS_W // _PAIR
_NBUF = 2
_LG = _D // 16


def _sc_bag_sum(xr, emb):
    mesh = plsc.VectorSubcoreMesh(
        core_axis_name="c", subcore_axis_name="s",
        num_cores=_NC, num_subcores=_NS)

    @functools.partial(
        pl.kernel,
        out_type=jax.ShapeDtypeStruct((_B, _D), jnp.float32),
        mesh=mesh,
        compiler_params=pltpu.CompilerParams(use_tc_tiling_on_sc=False),
        scratch_types=[
            pltpu.VMEM((_NCHUNK, _ROWS_CHUNK), jnp.int32),
            [pltpu.VMEM((_ROWS_CHUNK, _D), jnp.float32) for _ in range(_NBUF)],
            pltpu.VMEM((_BAGS_W, _D), jnp.float32),
            [pltpu.SemaphoreType.DMA for _ in range(_NBUF)],
        ],
    )
    def k(x_hbm, emb_hbm, out_hbm, idx_v, bufs, bag_v, sems):
        wid = lax.axis_index("s") * _NC + lax.axis_index("c")
        pltpu.sync_copy(x_hbm.at[pl.ds(wid * _NCHUNK, _NCHUNK)], idx_v)
        for s in range(_NBUF):
            pltpu.async_copy(emb_hbm.at[idx_v.at[s]], bufs[s], sems[s])

        def accum(j, buf):
            for h in range(_PAIR):
                r0 = h * _H
                acc = [buf[r0, pl.ds(l * 16, 16)] for l in range(_LG)]
                for r in range(1, _H):
                    for l in range(_LG):
                        acc[l] = acc[l] + buf[r0 + r, pl.ds(l * 16, 16)]
                for l in range(_LG):
                    bag_v[j * _PAIR + h, pl.ds(l * 16, 16)] = acc[l]

        @pl.loop(0, _NCHUNK, step=_NBUF)
        def _(jbase):
            for s in range(_NBUF):
                j = jbase + s
                pltpu.make_async_copy(
                    emb_hbm.at[idx_v.at[j]], bufs[s], sems[s]).wait()
                accum(j, bufs[s])
                nxt = j + _NBUF

                @pl.when(nxt < _NCHUNK)
                def _():
                    pltpu.async_copy(
                        emb_hbm.at[idx_v.at[nxt]], bufs[s], sems[s])

        pltpu.sync_copy(bag_v, out_hbm.at[pl.ds(wid * _BAGS_W, _BAGS_W)])

    return k(xr, emb)


def _tc_head(bag, wp, bp):
    tb = 1024

    def body(bag_ref, w_ref, b_ref, out_ref):
        y = jnp.dot(bag_ref[...], w_ref[...],
                    preferred_element_type=jnp.float32) + b_ref[...]
        m = jnp.max(y, axis=1, keepdims=True)
        e = jnp.exp(y - m)
        out_ref[...] = e / jnp.sum(e, axis=1, keepdims=True)

    return pl.pallas_call(
        body,
        grid=(_B // tb,),
        in_specs=[
            pl.BlockSpec((tb, _D), lambda i: (i, 0)),
            pl.BlockSpec((_D, 128), lambda i: (0, 0)),
            pl.BlockSpec((1, 128), lambda i: (0, 0)),
        ],
        out_specs=pl.BlockSpec((tb, 128), lambda i: (i, 0)),
        out_shape=jax.ShapeDtypeStruct((_B, 128), jnp.float32),
    )(bag, wp, bp)


def kernel(x, emb, W, b):
    x = x.astype(jnp.int32)
    xr = x.reshape(_NW * _NCHUNK, _ROWS_CHUNK)
    emb2 = jax.lax.optimization_barrier(emb.reshape(_VOCAB // 2, 2 * _D))
    bag = _sc_bag_sum(xr, emb2.reshape(_VOCAB, _D))
    wp = jnp.zeros((_D, 128), jnp.float32).at[:, :_OUT].set(W.T * (1.0 / _H))
    bp = jnp.full((1, 128), -1e30, jnp.float32).at[0, :_OUT].set(b)
    out = _tc_head(bag, wp, bp)
    return out[:, :_OUT]


# speedup vs baseline: 3.0091x; 1.2067x over previous
import functools

import jax
import jax.numpy as jnp
from jax import lax
from jax.experimental import pallas as pl
from jax.experimental.pallas import tpu as pltpu
from jax.experimental.pallas import tpu_sc as plsc

_VOCAB = 1000000
_D = 64
_OUT = 100
_B = 16384
_H = 50

_NC, _NS = 2, 16
_NW = _NC * _NS
_BAGS_W = _B // _NW
_PAIR = 2
_ROWS_CHUNK = _PAIR * _H
_NCHUNK = _BAGS_W // _PAIR
_NBUF = 2
_LG = _D // 16
_FLUSH = 32


def _tc_pairpack(embT):
    cb = 4096
    grid = (_VOCAB + cb - 1) // cb

    def body(in_ref, eye_ref, out_ref):
        t = jax.lax.dot_general(
            in_ref[...], eye_ref[...],
            dimension_numbers=(((0,), (0,)), ((), ())),
            preferred_element_type=jnp.float32)
        out_ref[...] = jnp.concatenate([t, t], axis=1)

    return pl.pallas_call(
        body,
        grid=(grid,),
        in_specs=[
            pl.BlockSpec((_D, cb), lambda i: (0, i)),
            pl.BlockSpec((_D, _D), lambda i: (0, 0)),
        ],
        out_specs=pl.BlockSpec((cb, 2 * _D), lambda i: (i, 0)),
        out_shape=jax.ShapeDtypeStruct((_VOCAB, 2 * _D), jnp.float32),
    )(embT, jnp.eye(_D, dtype=jnp.float32))


def _sc_bag_sum(xr, emb2):
    mesh = plsc.VectorSubcoreMesh(
        core_axis_name="c", subcore_axis_name="s",
        num_cores=_NC, num_subcores=_NS)

    @functools.partial(
        pl.kernel,
        out_type=jax.ShapeDtypeStruct((_B, _D), jnp.float32),
        mesh=mesh,
        compiler_params=pltpu.CompilerParams(
            use_tc_tiling_on_sc=True, needs_layout_passes=False),
        scratch_types=[
            pltpu.VMEM((_NCHUNK, _ROWS_CHUNK), jnp.int32),
            [pltpu.VMEM((_ROWS_CHUNK, 2 * _D), jnp.float32)
             for _ in range(_NBUF)],
            pltpu.VMEM((_PAIR * _FLUSH, _D), jnp.float32),
            [pltpu.SemaphoreType.DMA for _ in range(_NBUF)],
        ],
    )
    def k(x_hbm, emb_hbm, out_hbm, idx_v, bufs, stag, sems):
        wid = lax.axis_index("s") * _NC + lax.axis_index("c")
        pltpu.sync_copy(x_hbm.at[pl.ds(wid * _NCHUNK, _NCHUNK)], idx_v)

        for s in range(_NBUF):
            pltpu.async_copy(
                emb_hbm.at[idx_v.at[s].at[pl.ds(0, _ROWS_CHUNK)]],
                bufs[s], sems[s])

        def accum(j, buf):
            srow = _PAIR * lax.rem(j, _FLUSH)
            for h in range(_PAIR):
                r0 = h * _H
                acc = [buf[r0, pl.ds(l * 16, 16)] for l in range(_LG)]
                for r in range(1, _H):
                    for l in range(_LG):
                        acc[l] = acc[l] + buf[r0 + r, pl.ds(l * 16, 16)]
                for l in range(_LG):
                    stag[srow + h, pl.ds(l * 16, 16)] = acc[l]

        @pl.loop(0, _NCHUNK, step=_NBUF)
        def _(jbase):
            for s in range(_NBUF):
                j = jbase + s
                pltpu.make_async_copy(
                    emb_hbm.at[idx_v.at[j].at[pl.ds(0, _ROWS_CHUNK)]],
                    bufs[s], sems[s]).wait()
                accum(j, bufs[s])
                nxt = j + _NBUF

                @pl.when(nxt < _NCHUNK)
                def _():
                    pltpu.async_copy(
                        emb_hbm.at[idx_v.at[nxt].at[pl.ds(0, _ROWS_CHUNK)]],
                        bufs[s], sems[s])

            @pl.when(lax.rem(jbase, _FLUSH) == _FLUSH - _NBUF)
            def _():
                base = pl.multiple_of(
                    wid * _BAGS_W + _PAIR * (jbase - (_FLUSH - _NBUF)), 64)
                pltpu.sync_copy(
                    stag, out_hbm.at[pl.ds(base, _PAIR * _FLUSH)])

    return k(xr, emb2)


def _tc_head(bag, wp, bp):
    tb = 1024

    def body(bag_ref, w_ref, b_ref, out_ref):
        y = jnp.dot(bag_ref[...], w_ref[...],
                    preferred_element_type=jnp.float32) + b_ref[...]
        m = jnp.max(y, axis=1, keepdims=True)
        e = jnp.exp(y - m)
        out_ref[...] = e / jnp.sum(e, axis=1, keepdims=True)

    return pl.pallas_call(
        body,
        grid=(_B // tb,),
        in_specs=[
            pl.BlockSpec((tb, _D), lambda i: (i, 0)),
            pl.BlockSpec((_D, 128), lambda i: (0, 0)),
            pl.BlockSpec((1, 128), lambda i: (0, 0)),
        ],
        out_specs=pl.BlockSpec((tb, 128), lambda i: (i, 0)),
        out_shape=jax.ShapeDtypeStruct((_B, 128), jnp.float32),
    )(bag, wp, bp)


def kernel(x, emb, W, b):
    x = x.astype(jnp.int32)
    xr = x.reshape(_B * _H // _ROWS_CHUNK, _ROWS_CHUNK)
    emb2 = _tc_pairpack(emb.T)
    bag = _sc_bag_sum(xr, emb2)
    wp = jnp.zeros((_D, 128), jnp.float32).at[:, :_OUT].set(W.T * (1.0 / _H))
    bp = jnp.full((1, 128), -1e30, jnp.float32).at[0, :_OUT].set(b)
    out = _tc_head(bag, wp, bp)
    return out[:, :_OUT]
